# Initial kernel scaffold; baseline (speedup 1.0000x reference)
import jax
import jax.numpy as jnp
from jax.experimental import pallas as pl


def kernel(query, keys, values):
    raise NotImplementedError("write your pallas kernel here")


# speedup vs baseline: 1.7145x; 1.7145x over previous
import functools

import jax
import jax.numpy as jnp
from jax import lax
from jax.experimental import pallas as pl
from jax.experimental.pallas import tpu as pltpu
from jax.experimental.pallas import tpu_sc as plsc

Q = 1024
D = 64
N = 100000
KB = 2000
NB = N // KB


def _normalize(x, eps=1e-12):
    n = jnp.sqrt(jnp.sum(x * x, axis=-1, keepdims=True))
    return x / jnp.maximum(n, eps)


def _stats_body(q_ref, k_ref, idx_out, var_out, qn_s, rmax_s, ridx_s,
                sum_s, ssq_s):
    i = pl.program_id(0)

    @pl.when(i == 0)
    def _init():
        qn_s[...] = _normalize(q_ref[...])
        rmax_s[...] = jnp.full((Q, 1), -jnp.inf, jnp.float32)
        ridx_s[...] = jnp.zeros((Q, 1), jnp.int32)
        sum_s[...] = jnp.zeros((Q, 1), jnp.float32)
        ssq_s[...] = jnp.zeros((Q, 1), jnp.float32)

    kn = _normalize(k_ref[...])
    sim = lax.dot_general(qn_s[...], kn, (((1,), (1,)), ((), ())),
                          preferred_element_type=jnp.float32)

    bmax = jnp.max(sim, axis=1, keepdims=True)
    col = lax.broadcasted_iota(jnp.int32, (Q, KB), 1) + i * KB
    bidx = jnp.min(jnp.where(sim == bmax, col, jnp.int32(2**30)),
                   axis=1, keepdims=True)

    upd = bmax > rmax_s[...]
    ridx_s[...] = jnp.where(upd, bidx, ridx_s[...])
    rmax_s[...] = jnp.maximum(bmax, rmax_s[...])
    sum_s[...] += jnp.sum(sim, axis=1, keepdims=True)
    ssq_s[...] += jnp.sum(sim * sim, axis=1, keepdims=True)

    @pl.when(i == NB - 1)
    def _fin():
        idx_out[...] = ridx_s[...]
        s = sum_s[...]
        ss = ssq_s[...]
        var_rows = (ss - s * s / N) / (N - 1)
        var_out[...] = jnp.full((1, 1), jnp.mean(var_rows), jnp.float32)


def _topk_stats(query, keys, interpret=False):
    return pl.pallas_call(
        _stats_body,
        grid=(NB,),
        in_specs=[
            pl.BlockSpec((Q, D), lambda i: (0, 0)),
            pl.BlockSpec((KB, D), lambda i: (i, 0)),
        ],
        out_specs=[
            pl.BlockSpec((Q, 1), lambda i: (0, 0)),
            pl.BlockSpec((1, 1), lambda i: (0, 0)),
        ],
        out_shape=[
            jax.ShapeDtypeStruct((Q, 1), jnp.int32),
            jax.ShapeDtypeStruct((1, 1), jnp.float32),
        ],
        scratch_shapes=[
            pltpu.VMEM((Q, D), jnp.float32),
            pltpu.VMEM((Q, 1), jnp.float32),
            pltpu.VMEM((Q, 1), jnp.int32),
            pltpu.VMEM((Q, 1), jnp.float32),
            pltpu.VMEM((Q, 1), jnp.float32),
        ],
        interpret=interpret,
    )(query, keys)


def _sc_gather_pairs(values2, idx):
    info = plsc.get_sparse_core_info()
    nw = info.num_cores * info.num_subcores
    b_per_w = Q // nw
    mesh = plsc.VectorSubcoreMesh(core_axis_name="c", subcore_axis_name="s")

    @functools.partial(
        pl.kernel, mesh=mesh,
        out_type=jax.ShapeDtypeStruct((Q, 2 * D), jnp.float32),
        scratch_types=[
            pltpu.VMEM((b_per_w,), jnp.int32),
            pltpu.VMEM((b_per_w,), jnp.int32),
            pltpu.VMEM((b_per_w, 2 * D), jnp.float32),
            pltpu.SemaphoreType.DMA,
        ],
    )
    def gather_k(table_hbm, idx_hbm, out_hbm, idx_v, pair_v, rows_v, sem):
        wid = lax.axis_index("s") * info.num_cores + lax.axis_index("c")
        base = wid * b_per_w
        pltpu.sync_copy(idx_hbm.at[pl.ds(base, b_per_w)], idx_v)
        for c in range(b_per_w // 16):
            sl = pl.ds(c * 16, 16)
            pair_v[sl] = lax.shift_right_logical(idx_v[sl], 1)
        pltpu.async_copy(table_hbm.at[pair_v], rows_v, sem).wait()
        pltpu.sync_copy(rows_v, out_hbm.at[pl.ds(base, b_per_w)])

    return gather_k(values2, idx)


def _half_select_body(rows_ref, idx_ref, out_ref):
    odd = (idx_ref[...] & 1) == 1
    lo = rows_ref[:, :D]
    hi = rows_ref[:, D:]
    out_ref[...] = jnp.where(odd, hi, lo)


def _half_select(rows, idx):
    return pl.pallas_call(
        _half_select_body,
        out_shape=jax.ShapeDtypeStruct((Q, D), jnp.float32),
    )(rows, idx)


def kernel(query, keys, values):
    query = query.astype(jnp.float32)
    keys = keys.astype(jnp.float32)
    values = values.astype(jnp.float32)
    idx, var = _topk_stats(query, keys)
    rows = _sc_gather_pairs(values.reshape(N // 2, 2 * D), idx.reshape(Q))
    retrieved = _half_select(rows, idx)
    return (retrieved, var.reshape(()))


# speedup vs baseline: 1.8382x; 1.0721x over previous
import functools

import jax
import jax.numpy as jnp
from jax import lax
from jax.experimental import pallas as pl
from jax.experimental.pallas import tpu as pltpu
from jax.experimental.pallas import tpu_sc as plsc

Q = 1024
D = 64
N = 100000
KB = 2000
NB = N // KB


def _normalize(x, eps=1e-12):
    n = jnp.sqrt(jnp.sum(x * x, axis=-1, keepdims=True))
    return x / jnp.maximum(n, eps)


def _stats_body(q_ref, k_ref, idx_out, var_out, qn_s, rmax_s, ridx_s,
                ksum_s, gram_s):
    i = pl.program_id(0)

    @pl.when(i == 0)
    def _init():
        qn_s[...] = _normalize(q_ref[...])
        rmax_s[...] = jnp.full((Q, 1), -jnp.inf, jnp.float32)
        ridx_s[...] = jnp.zeros((Q, 1), jnp.int32)
        ksum_s[...] = jnp.zeros((1, D), jnp.float32)
        gram_s[...] = jnp.zeros((D, D), jnp.float32)

    kn = _normalize(k_ref[...])
    sim = lax.dot_general(qn_s[...], kn, (((1,), (1,)), ((), ())),
                          preferred_element_type=jnp.float32)

    bmax = jnp.max(sim, axis=1, keepdims=True)
    col = lax.broadcasted_iota(jnp.int32, (Q, KB), 1) + i * KB
    bidx = jnp.min(jnp.where(sim == bmax, col, jnp.int32(2**30)),
                   axis=1, keepdims=True)

    upd = bmax > rmax_s[...]
    ridx_s[...] = jnp.where(upd, bidx, ridx_s[...])
    rmax_s[...] = jnp.maximum(bmax, rmax_s[...])
    ksum_s[...] += jnp.sum(kn, axis=0, keepdims=True)
    gram_s[...] += lax.dot_general(kn, kn, (((0,), (0,)), ((), ())),
                                   preferred_element_type=jnp.float32)

    @pl.when(i == NB - 1)
    def _fin():
        idx_out[...] = ridx_s[...]
        qn = qn_s[...]
        s = lax.dot_general(qn, ksum_s[...], (((1,), (1,)), ((), ())),
                            preferred_element_type=jnp.float32)
        qg = lax.dot_general(qn, gram_s[...], (((1,), (0,)), ((), ())),
                             preferred_element_type=jnp.float32)
        ss = jnp.sum(qg * qn, axis=1, keepdims=True)
        var_rows = (ss - s * s / N) / (N - 1)
        var_out[...] = jnp.full((1, 1), jnp.mean(var_rows), jnp.float32)


def _topk_stats(query, keys, interpret=False):
    return pl.pallas_call(
        _stats_body,
        grid=(NB,),
        in_specs=[
            pl.BlockSpec((Q, D), lambda i: (0, 0)),
            pl.BlockSpec((KB, D), lambda i: (i, 0)),
        ],
        out_specs=[
            pl.BlockSpec((Q, 1), lambda i: (0, 0)),
            pl.BlockSpec((1, 1), lambda i: (0, 0)),
        ],
        out_shape=[
            jax.ShapeDtypeStruct((Q, 1), jnp.int32),
            jax.ShapeDtypeStruct((1, 1), jnp.float32),
        ],
        scratch_shapes=[
            pltpu.VMEM((Q, D), jnp.float32),
            pltpu.VMEM((Q, 1), jnp.float32),
            pltpu.VMEM((Q, 1), jnp.int32),
            pltpu.VMEM((1, D), jnp.float32),
            pltpu.VMEM((D, D), jnp.float32),
        ],
        interpret=interpret,
    )(query, keys)


def _sc_gather_pairs(values2, idx):
    info = plsc.get_sparse_core_info()
    nw = info.num_cores * info.num_subcores
    b_per_w = Q // nw
    mesh = plsc.VectorSubcoreMesh(core_axis_name="c", subcore_axis_name="s")

    @functools.partial(
        pl.kernel, mesh=mesh,
        out_type=jax.ShapeDtypeStruct((Q, 2 * D), jnp.float32),
        scratch_types=[
            pltpu.VMEM((b_per_w,), jnp.int32),
            pltpu.VMEM((b_per_w,), jnp.int32),
            pltpu.VMEM((b_per_w, 2 * D), jnp.float32),
            pltpu.SemaphoreType.DMA,
        ],
    )
    def gather_k(table_hbm, idx_hbm, out_hbm, idx_v, pair_v, rows_v, sem):
        wid = lax.axis_index("s") * info.num_cores + lax.axis_index("c")
        base = wid * b_per_w
        pltpu.sync_copy(idx_hbm.at[pl.ds(base, b_per_w)], idx_v)
        for c in range(b_per_w // 16):
            sl = pl.ds(c * 16, 16)
            pair_v[sl] = lax.shift_right_logical(idx_v[sl], 1)
        pltpu.async_copy(table_hbm.at[pair_v], rows_v, sem).wait()
        pltpu.sync_copy(rows_v, out_hbm.at[pl.ds(base, b_per_w)])

    return gather_k(values2, idx)


def _half_select_body(rows_ref, idx_ref, out_ref):
    odd = (idx_ref[...] & 1) == 1
    lo = rows_ref[:, :D]
    hi = rows_ref[:, D:]
    out_ref[...] = jnp.where(odd, hi, lo)


def _half_select(rows, idx):
    return pl.pallas_call(
        _half_select_body,
        out_shape=jax.ShapeDtypeStruct((Q, D), jnp.float32),
    )(rows, idx)


def kernel(query, keys, values):
    query = query.astype(jnp.float32)
    keys = keys.astype(jnp.float32)
    values = values.astype(jnp.float32)
    idx, var = _topk_stats(query, keys)
    rows = _sc_gather_pairs(values.reshape(N // 2, 2 * D), idx.reshape(Q))
    retrieved = _half_select(rows, idx)
    return (retrieved, var.reshape(()))
